# baseline (speedup 1.0000x reference)
import functools

import jax
import jax.numpy as jnp
from jax import lax
from jax.experimental import pallas as pl
from jax.experimental.pallas import tpu as pltpu
from jax.experimental.pallas import tpu_sc as plsc

B = 16384
EDIM = 16
NT = 4
NC, NS = 2, 16
NW = NC * NS
ROWS_PER_W = B // NW
CHUNK = 128
NCHUNK = ROWS_PER_W // CHUNK
H1 = 256
H2 = 128
BN_EPS_ = 1e-5


def _sc_gather(idx_r, e0, e1, e2, e3):
    mesh = plsc.VectorSubcoreMesh(core_axis_name="c", subcore_axis_name="s")

    @functools.partial(
        pl.kernel,
        mesh=mesh,
        compiler_params=pltpu.CompilerParams(use_tc_tiling_on_sc=False),
        out_type=jax.ShapeDtypeStruct((NW, NT, ROWS_PER_W, EDIM), jnp.float32),
        scratch_types=[
            pltpu.VMEM((NT, NCHUNK, CHUNK), jnp.int32),
            pltpu.VMEM((NT, ROWS_PER_W, EDIM), jnp.float32),
            pltpu.SemaphoreType.DMA,
        ],
    )
    def k(idx_hbm, t0, t1, t2, t3, out_hbm, idx_v, rows_v, sem):
        wid = lax.axis_index("s") * NC + lax.axis_index("c")
        pltpu.sync_copy(idx_hbm.at[wid], idx_v)
        cps = []
        for t, tab in enumerate((t0, t1, t2, t3)):
            for j in range(NCHUNK):
                cps.append(pltpu.async_copy(
                    tab.at[idx_v.at[t, j]],
                    rows_v.at[t, pl.ds(j * CHUNK, CHUNK)],
                    sem))
        for cp in cps:
            cp.wait()
        pltpu.sync_copy(rows_v, out_hbm.at[wid])

    return k(idx_r, e0, e1, e2, e3)


def _tc_mlp(xg, xcont, W1, b1, W2, b2, gamma, beta):
    BLK = ROWS_PER_W

    def body(xg_ref, xc_ref, w1_ref, b1_ref, w2_ref, b2_ref, g_ref, bt_ref,
             out_ref):
        inv = 1.0 / (1.0 + BN_EPS_) ** 0.5
        x2 = xc_ref[...] * (g_ref[0, 0] * inv) + bt_ref[0, 0]
        h = x2 * w1_ref[:, EDIM * NT:EDIM * NT + 1].T + b1_ref[...]
        for t in range(NT):
            h = h + lax.dot_general(
                xg_ref[0, t], w1_ref[:, t * EDIM:(t + 1) * EDIM],
                (((1,), (1,)), ((), ())),
                preferred_element_type=jnp.float32,
                precision=lax.Precision.HIGHEST)
        out_ref[...] = lax.dot_general(
            h, w2_ref[...], (((1,), (1,)), ((), ())),
            preferred_element_type=jnp.float32,
            precision=lax.Precision.HIGHEST) + b2_ref[...]

    return pl.pallas_call(
        body,
        grid=(NW,),
        in_specs=[
            pl.BlockSpec((1, NT, BLK, EDIM), lambda i: (i, 0, 0, 0)),
            pl.BlockSpec((BLK, 1), lambda i: (i, 0)),
            pl.BlockSpec((H1, EDIM * NT + 1), lambda i: (0, 0)),
            pl.BlockSpec((1, H1), lambda i: (0, 0)),
            pl.BlockSpec((H2, H1), lambda i: (0, 0)),
            pl.BlockSpec((1, H2), lambda i: (0, 0)),
            pl.BlockSpec((1, 1), lambda i: (0, 0)),
            pl.BlockSpec((1, 1), lambda i: (0, 0)),
        ],
        out_specs=pl.BlockSpec((BLK, H2), lambda i: (i, 0)),
        out_shape=jax.ShapeDtypeStruct((B, H2), jnp.float32),
    )(xg, xcont, W1, b1, W2, b2, gamma, beta)


def kernel(x_categorical, x_continuous, emb0, emb1, emb2, emb3,
           W1, b1, W2, b2, gamma, beta):
    idx_r = (x_categorical.astype(jnp.int32)
             .reshape(NW, ROWS_PER_W, NT)
             .swapaxes(1, 2)
             .reshape(NW, NT, NCHUNK, CHUNK))
    xg = _sc_gather(idx_r, emb0, emb1, emb2, emb3)
    out = _tc_mlp(xg, x_continuous, W1, b1.reshape(1, H1), W2,
                  b2.reshape(1, H2), gamma.reshape(1, 1), beta.reshape(1, 1))
    return out
